# baseline (speedup 1.0000x reference)
import functools
import math

import jax
import jax.numpy as jnp
from jax import lax
from jax.experimental import pallas as pl
from jax.experimental.pallas import tpu as pltpu
from jax.experimental.pallas import tpu_sc as plsc

B = 16384
D = 128
NC = 2
NS = 16
NW = NC * NS
BPW = B // NW
CH = 64
NCHUNK = BPW // CH
SCALE = math.sqrt(D)

_mesh = plsc.VectorSubcoreMesh(core_axis_name="c", subcore_axis_name="s")


@functools.partial(
    pl.kernel,
    mesh=_mesh,
    out_type=jax.ShapeDtypeStruct((B, D), jnp.float32),
    scratch_types=(
        [pltpu.VMEM((BPW,), jnp.int32)]
        + [pltpu.VMEM((CH, D), jnp.float32) for _ in range(NCHUNK)]
        + [pltpu.VMEM((CH, D), jnp.float32) for _ in range(2)]
        + [pltpu.SemaphoreType.DMA for _ in range(NCHUNK)]
        + [pltpu.SemaphoreType.DMA for _ in range(2)]
        + [pltpu.SemaphoreType.DMA for _ in range(NCHUNK + 1)]
    ),
)
def _distmult_sc(node_hbm, idx_hbm, table_hbm, out_hbm, idx_v, *bufs):
    rows = bufs[:NCHUNK]
    node = bufs[NCHUNK:NCHUNK + 2]
    sg = bufs[NCHUNK + 2:2 * NCHUNK + 2]
    sn = bufs[2 * NCHUNK + 2:2 * NCHUNK + 4]
    so = bufs[2 * NCHUNK + 4:]
    wid = lax.axis_index("s") * NC + lax.axis_index("c")
    base = wid * BPW

    pltpu.sync_copy(idx_hbm.at[pl.ds(base, CH)], idx_v.at[pl.ds(0, CH)])
    gflight = [None] * NCHUNK
    nflight = [None] * NCHUNK
    gflight[0] = pltpu.async_copy(table_hbm.at[idx_v.at[pl.ds(0, CH)]],
                                  rows[0], sg[0])
    nflight[0] = pltpu.async_copy(node_hbm.at[pl.ds(base, CH)], node[0], sn[0])
    pltpu.sync_copy(idx_hbm.at[pl.ds(base + CH, BPW - CH)],
                    idx_v.at[pl.ds(CH, BPW - CH)])
    for c in range(1, NCHUNK):
        gflight[c] = pltpu.async_copy(table_hbm.at[idx_v.at[pl.ds(c * CH, CH)]],
                                      rows[c], sg[c])
    nflight[1] = pltpu.async_copy(node_hbm.at[pl.ds(base + CH, CH)],
                                  node[1], sn[1])
    outflight = []
    for c in range(NCHUNK):
        gflight[c].wait()
        nflight[c].wait()
        last = c == NCHUNK - 1
        spans = ((0, CH // 2), (CH // 2, CH // 2)) if last else ((0, CH),)
        for k, (r0, nr) in enumerate(spans):

            def row_body(r, _, c=c):
                for i in range(D // 16):
                    sl = pl.ds(i * 16, 16)
                    rows[c][r, sl] = rows[c][r, sl] * (node[c % 2][r, sl] * SCALE)
                return 0

            lax.fori_loop(r0, r0 + nr, row_body, 0)
            sem = so[NCHUNK] if (last and k == 1) else so[c]
            outflight.append(pltpu.async_copy(
                rows[c].at[pl.ds(r0, nr)],
                out_hbm.at[pl.ds(base + c * CH + r0, nr)], sem))
        if c + 2 < NCHUNK:
            nflight[c + 2] = pltpu.async_copy(
                node_hbm.at[pl.ds(base + (c + 2) * CH, CH)], node[c % 2],
                sn[c % 2])
    for cp in outflight:
        cp.wait()


def kernel(node_emb, relation, rela_emb_weight):
    idx = relation.astype(jnp.int32)
    return _distmult_sc(node_emb, idx, rela_emb_weight)


# speedup vs baseline: 1.0356x; 1.0356x over previous
import functools
import math

import jax
import jax.numpy as jnp
from jax import lax
from jax.experimental import pallas as pl
from jax.experimental.pallas import tpu as pltpu
from jax.experimental.pallas import tpu_sc as plsc

B = 16384
D = 128
NC = 2
NS = 16
NW = NC * NS
BPW = B // NW
CH = 128
NCHUNK = BPW // CH
SCALE = math.sqrt(D)

_mesh = plsc.VectorSubcoreMesh(core_axis_name="c", subcore_axis_name="s")


@functools.partial(
    pl.kernel,
    mesh=_mesh,
    out_type=jax.ShapeDtypeStruct((B, D), jnp.float32),
    scratch_types=[
        pltpu.VMEM((BPW,), jnp.int32),
        pltpu.VMEM((CH, D), jnp.float32),
        pltpu.VMEM((CH, D), jnp.float32),
        pltpu.VMEM((CH, D), jnp.float32),
        pltpu.VMEM((CH, D), jnp.float32),
        pltpu.VMEM((CH, D), jnp.float32),
        pltpu.VMEM((CH, D), jnp.float32),
        pltpu.SemaphoreType.DMA,
        pltpu.SemaphoreType.DMA,
        pltpu.SemaphoreType.DMA,
        pltpu.SemaphoreType.DMA,
        pltpu.SemaphoreType.DMA,
        pltpu.SemaphoreType.DMA,
        pltpu.SemaphoreType.DMA,
        pltpu.SemaphoreType.DMA,
        pltpu.SemaphoreType.DMA,
        pltpu.SemaphoreType.DMA,
        pltpu.SemaphoreType.DMA,
    ],
)
def _distmult_sc(node_hbm, idx_hbm, table_hbm, out_hbm,
                 idx_v, rows0, rows1, rows2, rows3, node0, node1,
                 sg0, sg1, sg2, sg3, sn0, sn1, so0, so1, so2, so3, so4):
    wid = lax.axis_index("s") * NC + lax.axis_index("c")
    base = wid * BPW
    rows = (rows0, rows1, rows2, rows3)
    node = (node0, node1)
    sg = (sg0, sg1, sg2, sg3)
    sn = (sn0, sn1)
    so = (so0, so1, so2, so3)

    pltpu.sync_copy(idx_hbm.at[pl.ds(base, CH)], idx_v.at[pl.ds(0, CH)])
    gflight = [None] * NCHUNK
    nflight = [None] * NCHUNK
    gflight[0] = pltpu.async_copy(table_hbm.at[idx_v.at[pl.ds(0, CH)]],
                                  rows[0], sg[0])
    nflight[0] = pltpu.async_copy(node_hbm.at[pl.ds(base, CH)], node[0], sn[0])
    pltpu.sync_copy(idx_hbm.at[pl.ds(base + CH, BPW - CH)],
                    idx_v.at[pl.ds(CH, BPW - CH)])
    for c in range(1, NCHUNK):
        gflight[c] = pltpu.async_copy(table_hbm.at[idx_v.at[pl.ds(c * CH, CH)]],
                                      rows[c], sg[c])
    nflight[1] = pltpu.async_copy(node_hbm.at[pl.ds(base + CH, CH)],
                                  node[1], sn[1])
    outflight = []
    for c in range(NCHUNK):
        gflight[c].wait()
        nflight[c].wait()
        last = c == NCHUNK - 1
        spans = ((0, CH // 2), (CH // 2, CH // 2)) if last else ((0, CH),)
        for k, (r0, nr) in enumerate(spans):

            def row_body(r, _, c=c):
                for i in range(D // 16):
                    sl = pl.ds(i * 16, 16)
                    rows[c][r, sl] = rows[c][r, sl] * (node[c % 2][r, sl] * SCALE)
                return 0

            lax.fori_loop(r0, r0 + nr, row_body, 0)
            sem = so4 if (last and k == 1) else so[c]
            outflight.append(pltpu.async_copy(
                rows[c].at[pl.ds(r0, nr)],
                out_hbm.at[pl.ds(base + c * CH + r0, nr)], sem))
        if c + 2 < NCHUNK:
            nflight[c + 2] = pltpu.async_copy(
                node_hbm.at[pl.ds(base + (c + 2) * CH, CH)], node[c % 2],
                sn[c % 2])
    for cp in outflight:
        cp.wait()


def kernel(node_emb, relation, rela_emb_weight):
    idx = relation.astype(jnp.int32)
    return _distmult_sc(node_emb, idx, rela_emb_weight)
